# Initial kernel scaffold; baseline (speedup 1.0000x reference)
import functools

import jax
import jax.numpy as jnp
from jax import lax
from jax.experimental import pallas as pl
from jax.experimental.pallas import tpu as pltpu

N = 10000
E = 320000
D = 128
NP = 10240

EGO_BI = 1024
EGO_BK = 256


def _ego_body(a_ref, b_ref, x_ref, o_ref, cnt_ref):
    k = pl.program_id(1)
    nk = pl.num_programs(1)
    counts = jnp.dot(a_ref[...], b_ref[...], preferred_element_type=jnp.float32)
    direct = a_ref[:, pl.ds(k * EGO_BK, EGO_BK)].astype(jnp.float32)
    tot = counts + direct
    i = pl.program_id(0)
    rows = i * EGO_BI + lax.broadcasted_iota(jnp.int32, (EGO_BI, EGO_BK), 0)
    cols = k * EGO_BK + lax.broadcasted_iota(jnp.int32, (EGO_BI, EGO_BK), 1)
    reach = jnp.where((tot > 0.5) | (rows == cols), 1.0, 0.0)
    contrib = jnp.dot(reach, x_ref[...], preferred_element_type=jnp.float32)
    rc = jnp.sum(reach, axis=1, keepdims=True)

    @pl.when(k == 0)
    def _():
        cnt_ref[...] = rc
        o_ref[...] = contrib

    @pl.when(k > 0)
    def _():
        cnt_ref[...] += rc
        o_ref[...] += contrib

    @pl.when(k == nk - 1)
    def _():
        o_ref[...] = o_ref[...] / cnt_ref[...]


def _ego_pallas(Bbf, x_pad):
    grid = (NP // EGO_BI, NP // EGO_BK)
    return pl.pallas_call(
        _ego_body,
        grid=grid,
        in_specs=[
            pl.BlockSpec((EGO_BI, NP), lambda i, k: (i, 0)),
            pl.BlockSpec((NP, EGO_BK), lambda i, k: (0, k)),
            pl.BlockSpec((EGO_BK, D), lambda i, k: (k, 0)),
        ],
        out_specs=pl.BlockSpec((EGO_BI, D), lambda i, k: (i, 0)),
        out_shape=jax.ShapeDtypeStruct((NP, D), jnp.float32),
        scratch_shapes=[pltpu.VMEM((EGO_BI, 1), jnp.float32)],
    )(Bbf, x_pad)


def _mid_body(x_ref, ego_ref, cutn_ref, cutd_ref, cosn_ref, cosd_ref,
              we_ref, be_ref, wc_ref, bc_ref, wco_ref, bco_ref, wg_ref, bg_ref,
              h_ref, glob_ref):
    x = x_ref[...]
    den = cutd_ref[...]
    has_nb = den > 0.0
    cut = jnp.where(has_nb, cutn_ref[...] / jnp.maximum(den, 1e-12), x)
    cosd = cosd_ref[...]
    cosine = jnp.where(has_nb, cosn_ref[...] / jnp.maximum(cosd, 1e-12), x)
    ego = ego_ref[...]
    he = jnp.dot(ego, we_ref[...].T, preferred_element_type=jnp.float32) + be_ref[...]
    hc = jnp.dot(cut, wc_ref[...].T, preferred_element_type=jnp.float32) + bc_ref[...]
    ho = jnp.dot(cosine, wco_ref[...].T, preferred_element_type=jnp.float32) + bco_ref[...]
    h_ref[...] = jnp.concatenate([he, hc, ho], axis=1)
    glob_ref[...] = jnp.dot(x, wg_ref[...].T, preferred_element_type=jnp.float32) + bg_ref[...]


def _mid_pallas(x, ego, cut_num, cut_den, cos_num, cos_den,
                W_ego, b_ego, W_cut, b_cut, W_cos, b_cos, W_glob, b_glob):
    BR = 1000
    grid = (N // BR,)
    row = pl.BlockSpec((BR, D), lambda i: (i, 0))
    rowc = pl.BlockSpec((BR, 1), lambda i: (i, 0))
    wspec = pl.BlockSpec((D, D), lambda i: (0, 0))
    bspec = pl.BlockSpec((1, D), lambda i: (0, 0))
    return pl.pallas_call(
        _mid_body,
        grid=grid,
        in_specs=[row, row, row, rowc, row, rowc,
                  wspec, bspec, wspec, bspec, wspec, bspec, wspec, bspec],
        out_specs=[pl.BlockSpec((BR, 3 * D), lambda i: (i, 0)), row],
        out_shape=[jax.ShapeDtypeStruct((N, 3 * D), jnp.float32),
                   jax.ShapeDtypeStruct((N, D), jnp.float32)],
    )(x, ego, cut_num, cut_den, cos_num, cos_den,
      W_ego, b_ego.reshape(1, D), W_cut, b_cut.reshape(1, D),
      W_cos, b_cos.reshape(1, D), W_glob, b_glob.reshape(1, D))


def _tail_body(agg_ref, glob_ref, wfc_ref, bfc_ref, o_ref):
    comb = jnp.concatenate([jax.nn.relu(agg_ref[...]), glob_ref[...]], axis=1)
    logits = jnp.dot(comb, wfc_ref[...].T, preferred_element_type=jnp.float32) + bfc_ref[...]
    m = jnp.max(logits, axis=1, keepdims=True)
    s = logits - m
    lse = jnp.log(jnp.sum(jnp.exp(s), axis=1, keepdims=True))
    o_ref[...] = s - lse


def _tail_pallas(agg, glob, W_fc, b_fc):
    BR = 1000
    grid = (N // BR,)
    return pl.pallas_call(
        _tail_body,
        grid=grid,
        in_specs=[
            pl.BlockSpec((BR, 3 * D), lambda i: (i, 0)),
            pl.BlockSpec((BR, D), lambda i: (i, 0)),
            pl.BlockSpec((D, 4 * D), lambda i: (0, 0)),
            pl.BlockSpec((1, D), lambda i: (0, 0)),
        ],
        out_specs=pl.BlockSpec((BR, D), lambda i: (i, 0)),
        out_shape=jax.ShapeDtypeStruct((N, D), jnp.float32),
    )(agg, glob, W_fc, b_fc.reshape(1, D))


def kernel(x, edge_index, W_ego, b_ego, W_cut, b_cut, W_cos, b_cos,
           W_glob, b_glob, W_fc, b_fc):
    src = edge_index[0]
    dst = edge_index[1]

    B01 = jnp.zeros((NP, NP), jnp.float32).at[dst, src].set(1.0)
    Bbf = B01.astype(jnp.bfloat16)
    x_pad = jnp.pad(x, ((0, NP - N), (0, 0)))

    ego = _ego_pallas(Bbf, x_pad)[:N]

    ones = jnp.ones((E,), jnp.float32)
    cut_num = jax.ops.segment_sum(x[dst], src, num_segments=N)
    cut_den = jax.ops.segment_sum(ones, src, num_segments=N)

    nx = x / jnp.maximum(jnp.linalg.norm(x, axis=1, keepdims=True), 1e-12)
    cos = jnp.sum(nx[dst] * nx[src], axis=1)
    e = jnp.exp(cos)
    s = jax.ops.segment_sum(e, src, num_segments=N)
    cos_num = jax.ops.segment_sum(x[dst] * e[:, None], src, num_segments=N)

    h_all, glob = _mid_pallas(
        x, ego, cut_num, cut_den.reshape(N, 1), cos_num, s.reshape(N, 1),
        W_ego, b_ego, W_cut, b_cut, W_cos, b_cos, W_glob, b_glob)

    agg = jax.ops.segment_sum(h_all[src], dst, num_segments=N)

    return _tail_pallas(agg, glob, W_fc, b_fc)


# speedup vs baseline: 1.5656x; 1.5656x over previous
import functools

import jax
import jax.numpy as jnp
from jax import lax
from jax.experimental import pallas as pl
from jax.experimental.pallas import tpu as pltpu

N = 10000
E = 320000
D = 128
NP = 10240

EGO_BI = 1024
EGO_BK = 256


def _ego_body(a_ref, b_ref, x_ref, o_ref, cnt_ref):
    k = pl.program_id(1)
    nk = pl.num_programs(1)
    counts = jnp.dot(a_ref[...], b_ref[...], preferred_element_type=jnp.float32)
    direct = a_ref[:, pl.ds(k * EGO_BK, EGO_BK)].astype(jnp.float32)
    tot = counts + direct
    i = pl.program_id(0)
    rows = i * EGO_BI + lax.broadcasted_iota(jnp.int32, (EGO_BI, EGO_BK), 0)
    cols = k * EGO_BK + lax.broadcasted_iota(jnp.int32, (EGO_BI, EGO_BK), 1)
    reach = jnp.where((tot > 0.5) | (rows == cols), 1.0, 0.0)
    contrib = jnp.dot(reach, x_ref[...], preferred_element_type=jnp.float32)
    rc = jnp.sum(reach, axis=1, keepdims=True)

    @pl.when(k == 0)
    def _():
        cnt_ref[...] = rc
        o_ref[...] = contrib

    @pl.when(k > 0)
    def _():
        cnt_ref[...] += rc
        o_ref[...] += contrib

    @pl.when(k == nk - 1)
    def _():
        o_ref[...] = o_ref[...] / cnt_ref[...]


def _ego_pallas(Bbf, x_pad):
    grid = (NP // EGO_BI, NP // EGO_BK)
    return pl.pallas_call(
        _ego_body,
        grid=grid,
        in_specs=[
            pl.BlockSpec((EGO_BI, NP), lambda i, k: (i, 0)),
            pl.BlockSpec((NP, EGO_BK), lambda i, k: (0, k)),
            pl.BlockSpec((EGO_BK, D), lambda i, k: (k, 0)),
        ],
        out_specs=pl.BlockSpec((EGO_BI, D), lambda i, k: (i, 0)),
        out_shape=jax.ShapeDtypeStruct((NP, D), jnp.float32),
        scratch_shapes=[pltpu.VMEM((EGO_BI, 1), jnp.float32)],
    )(Bbf, Bbf, x_pad)


def _mid_body(x_ref, ego_ref, cutn_ref, cutd_ref, cosn_ref, cosd_ref,
              we_ref, be_ref, wc_ref, bc_ref, wco_ref, bco_ref, wg_ref, bg_ref,
              h_ref, glob_ref):
    x = x_ref[...]
    den = cutd_ref[...]
    has_nb = den > 0.0
    cut = jnp.where(has_nb, cutn_ref[...] / jnp.maximum(den, 1e-12), x)
    cosd = cosd_ref[...]
    cosine = jnp.where(has_nb, cosn_ref[...] / jnp.maximum(cosd, 1e-12), x)
    ego = ego_ref[...]
    he = jnp.dot(ego, we_ref[...].T, preferred_element_type=jnp.float32) + be_ref[...]
    hc = jnp.dot(cut, wc_ref[...].T, preferred_element_type=jnp.float32) + bc_ref[...]
    ho = jnp.dot(cosine, wco_ref[...].T, preferred_element_type=jnp.float32) + bco_ref[...]
    h_ref[...] = jnp.concatenate([he, hc, ho], axis=1)
    glob_ref[...] = jnp.dot(x, wg_ref[...].T, preferred_element_type=jnp.float32) + bg_ref[...]


def _mid_pallas(x, ego, cut_num, cut_den, cos_num, cos_den,
                W_ego, b_ego, W_cut, b_cut, W_cos, b_cos, W_glob, b_glob):
    BR = 1000
    grid = (N // BR,)
    row = pl.BlockSpec((BR, D), lambda i: (i, 0))
    rowc = pl.BlockSpec((BR, 1), lambda i: (i, 0))
    wspec = pl.BlockSpec((D, D), lambda i: (0, 0))
    bspec = pl.BlockSpec((1, D), lambda i: (0, 0))
    return pl.pallas_call(
        _mid_body,
        grid=grid,
        in_specs=[row, row, row, rowc, row, rowc,
                  wspec, bspec, wspec, bspec, wspec, bspec, wspec, bspec],
        out_specs=[pl.BlockSpec((BR, 3 * D), lambda i: (i, 0)), row],
        out_shape=[jax.ShapeDtypeStruct((N, 3 * D), jnp.float32),
                   jax.ShapeDtypeStruct((N, D), jnp.float32)],
    )(x, ego, cut_num, cut_den, cos_num, cos_den,
      W_ego, b_ego.reshape(1, D), W_cut, b_cut.reshape(1, D),
      W_cos, b_cos.reshape(1, D), W_glob, b_glob.reshape(1, D))


def _tail_body(agg_ref, glob_ref, wfc_ref, bfc_ref, o_ref):
    comb = jnp.concatenate([jax.nn.relu(agg_ref[...]), glob_ref[...]], axis=1)
    logits = jnp.dot(comb, wfc_ref[...].T, preferred_element_type=jnp.float32) + bfc_ref[...]
    m = jnp.max(logits, axis=1, keepdims=True)
    s = logits - m
    lse = jnp.log(jnp.sum(jnp.exp(s), axis=1, keepdims=True))
    o_ref[...] = s - lse


def _tail_pallas(agg, glob, W_fc, b_fc):
    BR = 1000
    grid = (N // BR,)
    return pl.pallas_call(
        _tail_body,
        grid=grid,
        in_specs=[
            pl.BlockSpec((BR, 3 * D), lambda i: (i, 0)),
            pl.BlockSpec((BR, D), lambda i: (i, 0)),
            pl.BlockSpec((D, 4 * D), lambda i: (0, 0)),
            pl.BlockSpec((1, D), lambda i: (0, 0)),
        ],
        out_specs=pl.BlockSpec((BR, D), lambda i: (i, 0)),
        out_shape=jax.ShapeDtypeStruct((N, D), jnp.float32),
    )(agg, glob, W_fc, b_fc.reshape(1, D))


def kernel(x, edge_index, W_ego, b_ego, W_cut, b_cut, W_cos, b_cos,
           W_glob, b_glob, W_fc, b_fc):
    src = edge_index[0]
    dst = edge_index[1]

    B01 = jnp.zeros((NP, NP), jnp.float32).at[dst, src].set(1.0)
    Bbf = B01.astype(jnp.bfloat16)
    x_pad = jnp.pad(x, ((0, NP - N), (0, 0)))

    ego = _ego_pallas(Bbf, x_pad)[:N]

    ones = jnp.ones((E,), jnp.float32)
    cut_num = jax.ops.segment_sum(x[dst], src, num_segments=N)
    cut_den = jax.ops.segment_sum(ones, src, num_segments=N)

    nx = x / jnp.maximum(jnp.linalg.norm(x, axis=1, keepdims=True), 1e-12)
    cos = jnp.sum(nx[dst] * nx[src], axis=1)
    e = jnp.exp(cos)
    s = jax.ops.segment_sum(e, src, num_segments=N)
    cos_num = jax.ops.segment_sum(x[dst] * e[:, None], src, num_segments=N)

    h_all, glob = _mid_pallas(
        x, ego, cut_num, cut_den.reshape(N, 1), cos_num, s.reshape(N, 1),
        W_ego, b_ego, W_cut, b_cut, W_cos, b_cos, W_glob, b_glob)

    agg = jax.ops.segment_sum(h_all[src], dst, num_segments=N)

    return _tail_pallas(agg, glob, W_fc, b_fc)
